# baseline (speedup 1.0000x reference)
import functools

import jax
import jax.numpy as jnp
from jax import lax
from jax.experimental import pallas as pl
from jax.experimental.pallas import tpu as pltpu
from jax.experimental.pallas import tpu_sc as plsc

F32 = jnp.float32
I32 = jnp.int32

KS = 5
KK = KS * KS
OUT = 16
N = 10000
E = 160000
NC = 2
NS = 16
NW = NC * NS
CH = 128
NCHUNK = 40
PE = CH * NCHUNK * NW
NCH0 = 46
NCH1 = 34
NCHMAX = max(NCH0, NCH1)
HW = KK * OUT
NPAD = 10112
ZR = NPAD // NS
NB = 2000


def _edge_prep_body(attr_ref, ei_ref, basis_ref, gidx_ref):
    a0 = attr_ref[0, :]
    a1 = attr_ref[1, :]
    v0 = a0 * float(KS - 1)
    v1 = a1 * float(KS - 1)
    lo0 = jnp.floor(v0)
    lo1 = jnp.floor(v1)
    f0 = v0 - lo0
    f1 = v1 - lo1
    lo0i = jnp.clip(lo0.astype(I32), 0, KS - 1)
    lo1i = jnp.clip(lo1.astype(I32), 0, KS - 1)
    hi0i = jnp.clip(lo0i + 1, 0, KS - 1)
    hi1i = jnp.clip(lo1i + 1, 0, KS - 1)
    g0 = f0
    g1 = f1
    src = ei_ref[0, :]
    base = src * KK
    basis_ref[0, :] = (1.0 - g0) * (1.0 - g1)
    basis_ref[1, :] = g0 * (1.0 - g1)
    basis_ref[2, :] = (1.0 - g0) * g1
    basis_ref[3, :] = g0 * g1
    gidx_ref[0, :] = base + lo0i + KS * lo1i
    gidx_ref[1, :] = base + hi0i + KS * lo1i
    gidx_ref[2, :] = base + lo0i + KS * hi1i
    gidx_ref[3, :] = base + hi0i + KS * hi1i


def _edge_prep(attr_t, ei_pad):
    be = 8192
    grid = PE // be
    return pl.pallas_call(
        _edge_prep_body,
        grid=(grid,),
        in_specs=[
            pl.BlockSpec((2, be), lambda i: (0, i)),
            pl.BlockSpec((2, be), lambda i: (0, i)),
        ],
        out_specs=[
            pl.BlockSpec((4, be), lambda i: (0, i)),
            pl.BlockSpec((4, be), lambda i: (0, i)),
        ],
        out_shape=[
            jax.ShapeDtypeStruct((4, PE), F32),
            jax.ShapeDtypeStruct((4, PE), I32),
        ],
    )(attr_t, ei_pad)


def _h_body(x_ref, w_ref, o_ref):
    o_ref[...] = jnp.dot(x_ref[...], w_ref[...], preferred_element_type=F32)


def _h_matmul(x, wflat):
    din = x.shape[1]
    return pl.pallas_call(
        _h_body,
        grid=(N // NB,),
        in_specs=[
            pl.BlockSpec((NB, din), lambda i: (i, 0)),
            pl.BlockSpec((din, HW), lambda i: (0, 0)),
        ],
        out_specs=pl.BlockSpec((NB, HW), lambda i: (i, 0)),
        out_shape=jax.ShapeDtypeStruct((N, HW), F32),
    )(x, wflat)


def _epi_body(p_ref, d_ref, x_ref, root_ref, b_ref, wn_ref, xo_ref, h_ref):
    agg = p_ref[0] + p_ref[1]
    deg = d_ref[0, :, 0:1] + d_ref[1, :, 0:1]
    agg = agg / jnp.maximum(deg, 1.0)
    lin = jnp.dot(x_ref[...], root_ref[...], preferred_element_type=F32)
    xo = jnp.maximum(agg + lin + b_ref[...], 0.0)
    xo_ref[...] = xo
    h_ref[...] = jnp.dot(xo, wn_ref[...], preferred_element_type=F32)


def _epilogue(p, d, x, root, b, wnext):
    din = x.shape[1]
    return pl.pallas_call(
        _epi_body,
        grid=(N // NB,),
        in_specs=[
            pl.BlockSpec((2, NB, OUT), lambda i: (0, i, 0)),
            pl.BlockSpec((2, NB, OUT), lambda i: (0, i, 0)),
            pl.BlockSpec((NB, din), lambda i: (i, 0)),
            pl.BlockSpec((din, OUT), lambda i: (0, 0)),
            pl.BlockSpec((1, OUT), lambda i: (0, 0)),
            pl.BlockSpec((OUT, HW), lambda i: (0, 0)),
        ],
        out_specs=[
            pl.BlockSpec((NB, OUT), lambda i: (i, 0)),
            pl.BlockSpec((NB, HW), lambda i: (i, 0)),
        ],
        out_shape=[
            jax.ShapeDtypeStruct((N, OUT), F32),
            jax.ShapeDtypeStruct((N, HW), F32),
        ],
    )(p, d, x, root, b.reshape(1, OUT), wnext)


def _final_body(p_ref, d_ref, x_ref, root_ref, b_ref, fcw_ref, fcb_ref, o_ref):
    agg = p_ref[0] + p_ref[1]
    deg = d_ref[0, :, 0:1] + d_ref[1, :, 0:1]
    agg = agg / jnp.maximum(deg, 1.0)
    lin = jnp.dot(x_ref[...], root_ref[...], preferred_element_type=F32)
    h = jnp.maximum(agg + lin + b_ref[...], 0.0)
    o = jnp.dot(h, fcw_ref[...], preferred_element_type=F32) + fcb_ref[...]
    o_ref[...] = jax.nn.sigmoid(o)


def _final(p, d, x, root, b, fc_w, fc_b):
    return pl.pallas_call(
        _final_body,
        grid=(N // NB,),
        in_specs=[
            pl.BlockSpec((2, NB, OUT), lambda i: (0, i, 0)),
            pl.BlockSpec((2, NB, OUT), lambda i: (0, i, 0)),
            pl.BlockSpec((NB, OUT), lambda i: (i, 0)),
            pl.BlockSpec((OUT, OUT), lambda i: (0, 0)),
            pl.BlockSpec((1, OUT), lambda i: (0, 0)),
            pl.BlockSpec((OUT, 1), lambda i: (0, 0)),
            pl.BlockSpec((1, 1), lambda i: (0, 0)),
        ],
        out_specs=pl.BlockSpec((NB, 1), lambda i: (i, 0)),
        out_shape=jax.ShapeDtypeStruct((N, 1), F32),
    )(p, d, x, root, b.reshape(1, OUT), fc_w, fc_b.reshape(1, 1))


@functools.cache
def _make_sc_pass(with_deg):
    mesh = plsc.VectorSubcoreMesh(core_axis_name="c", subcore_axis_name="s",
                                  num_cores=NC, num_subcores=NS)
    n_out = 2 if with_deg else 1
    out_type = [jax.ShapeDtypeStruct((NC, NPAD, OUT), F32)] * n_out
    scratch = [
        pltpu.VMEM_SHARED((NPAD, OUT), F32),
        pltpu.VMEM((NCHMAX, CH), I32),
        pltpu.VMEM((4, NCHMAX, CH), I32),
        pltpu.VMEM((4, NCHMAX, CH), F32),
        pltpu.VMEM((2, 4, CH, OUT), F32),
        pltpu.VMEM((2, CH, OUT), F32),
        pltpu.VMEM((ZR, OUT), F32),
        pltpu.SemaphoreType.DMA,
        pltpu.SemaphoreType.DMA,
        pltpu.SemaphoreType.DMA,
        pltpu.SemaphoreType.DMA,
    ]
    if with_deg:
        scratch.insert(1, pltpu.VMEM_SHARED((NPAD, OUT), F32))
        scratch.append(pltpu.VMEM((CH, OUT), F32))

    def body(*refs):
        if with_deg:
            (h_hbm, gidx_hbm, basis_hbm, dst_hbm, out_hbm, deg_hbm,
             acc_sh, deg_sh, dst_all, gidx_all, basis_all,
             rbuf, msg_v, zbuf, g0, g1, w0, w1, ones_v) = refs
        else:
            (h_hbm, gidx_hbm, basis_hbm, dst_hbm, out_hbm,
             acc_sh, dst_all, gidx_all, basis_all,
             rbuf, msg_v, zbuf, g0, g1, w0, w1) = refs
            deg_hbm = deg_sh = ones_v = None

        c = lax.axis_index("c")
        s = lax.axis_index("s")
        base_chunk = jnp.where(c == 0, s * NCH0, NS * NCH0 + s * NCH1)
        my_npair = jnp.where(c == 0, NCH0 // 2, NCH1 // 2)
        zero_row = jnp.zeros((OUT,), F32)

        def zero_body(j, carry):
            zbuf[j] = zero_row
            return carry

        lax.fori_loop(0, ZR, zero_body, 0)
        r0off = s * ZR
        pltpu.sync_copy(zbuf, acc_sh.at[pl.ds(r0off, ZR)])
        if with_deg:
            pltpu.sync_copy(zbuf, deg_sh.at[pl.ds(r0off, ZR)])
            one_row = jnp.ones((OUT,), F32)

            def one_body(j, carry):
                ones_v[j] = one_row
                return carry

            lax.fori_loop(0, CH, one_body, 0)

        pltpu.sync_copy(dst_hbm.at[pl.ds(base_chunk, NCHMAX)], dst_all)
        for si in range(4):
            pltpu.sync_copy(gidx_hbm.at[si, pl.ds(base_chunk, NCHMAX)],
                            gidx_all.at[si])
        for si in range(4):
            pltpu.sync_copy(basis_hbm.at[si, pl.ds(base_chunk, NCHMAX)],
                            basis_all.at[si])

        plsc.subcore_barrier()

        gsems = (g0, g1)
        wsems = (w0, w1)
        dummy = h_hbm.at[pl.ds(0, CH)]
        dummy16 = out_hbm.at[0, pl.ds(0, CH)]

        def fire_gather(p, cix):
            for si in range(4):
                pltpu.async_copy(h_hbm.at[gidx_all.at[si, cix]],
                                 rbuf.at[p, si], gsems[p])

        def drain_gather(p):
            for si in range(4):
                pltpu.make_async_copy(dummy, rbuf.at[p, si], gsems[p]).wait()

        def fire_scatter(p, cix):
            pltpu.async_copy(msg_v.at[p], acc_sh.at[dst_all.at[cix]],
                             wsems[p], add=True)
            if with_deg:
                pltpu.async_copy(ones_v, deg_sh.at[dst_all.at[cix]],
                                 wsems[p], add=True)

        def drain_scatter(p):
            pltpu.make_async_copy(dummy16, msg_v.at[p], wsems[p]).wait()
            if with_deg:
                pltpu.make_async_copy(dummy16, ones_v, wsems[p]).wait()

        def compute(p, cix):
            def comp(gi, carry2):
                j0 = gi * 16
                bvs = [basis_all[si, cix, pl.ds(j0, 16)] for si in range(4)]
                for jj in range(16):
                    j = j0 + jj
                    m = (bvs[0][jj] * rbuf[p, 0, j]
                         + bvs[1][jj] * rbuf[p, 1, j]
                         + bvs[2][jj] * rbuf[p, 2, j]
                         + bvs[3][jj] * rbuf[p, 3, j])
                    msg_v[p, j] = m
                return carry2

            lax.fori_loop(0, CH // 16, comp, 0)

        fire_gather(0, 0)

        def pair_body(gg, carry):
            a = 2 * gg
            b = a + 1
            fire_gather(1, b)
            drain_gather(0)

            @pl.when(gg >= 1)
            def _():
                drain_scatter(0)

            compute(0, a)
            fire_scatter(0, a)

            @pl.when(gg < my_npair - 1)
            def _():
                fire_gather(0, a + 2)

            drain_gather(1)

            @pl.when(gg >= 1)
            def _():
                drain_scatter(1)

            compute(1, b)
            fire_scatter(1, b)
            return carry

        lax.fori_loop(0, my_npair, pair_body, 0)
        drain_scatter(0)
        drain_scatter(1)

        plsc.subcore_barrier()

        pltpu.sync_copy(acc_sh.at[pl.ds(r0off, ZR)], zbuf)
        pltpu.sync_copy(zbuf, out_hbm.at[c, pl.ds(r0off, ZR)])
        if with_deg:
            pltpu.sync_copy(deg_sh.at[pl.ds(r0off, ZR)], zbuf)
            pltpu.sync_copy(zbuf, deg_hbm.at[c, pl.ds(r0off, ZR)])

    return pl.kernel(body, out_type=out_type, mesh=mesh,
                     scratch_types=scratch,
                     compiler_params=pltpu.CompilerParams(
                         use_tc_tiling_on_sc=False))


def kernel(x, edge_index, edge_attr, W1, root1, b1, W2, root2, b2,
           W3, root3, b3, fc_w, fc_b):
    src = edge_index[0]
    dst = edge_index[1]
    attr_t = jnp.pad(edge_attr.T, ((0, 0), (0, PE - E)))
    trash = N + (jnp.arange(PE - E, dtype=I32) % (NPAD - N))
    ei_pad = jnp.stack([
        jnp.pad(src, (0, PE - E)),
        jnp.concatenate([dst, trash]),
    ])
    basis4, gidx4 = _edge_prep(attr_t, ei_pad)
    dst3 = ei_pad[1].reshape(PE // CH, CH)
    gidx3 = gidx4.reshape(4, PE // CH, CH)
    basis3 = basis4.reshape(4, PE // CH, CH)

    def wt_flat(W):
        return jnp.transpose(W, (1, 0, 2)).reshape(W.shape[1], HW)

    wf1 = wt_flat(W1)
    wf2 = wt_flat(W2)
    wf3 = wt_flat(W3)

    h1 = _h_matmul(x, wf1).reshape(N * KK, OUT)
    p1, dcount = _make_sc_pass(True)(h1, gidx3, basis3, dst3)
    x1, h2 = _epilogue(p1, dcount, x, root1, b1, wf2)
    p2 = _make_sc_pass(False)(h2.reshape(N * KK, OUT), gidx3, basis3, dst3)
    p2 = p2[0] if isinstance(p2, (list, tuple)) else p2
    x2, h3 = _epilogue(p2, dcount, x1, root2, b2, wf3)
    p3 = _make_sc_pass(False)(h3.reshape(N * KK, OUT), gidx3, basis3, dst3)
    p3 = p3[0] if isinstance(p3, (list, tuple)) else p3
    out = _final(p3, dcount, x2, root3, b3, fc_w, fc_b)
    return out.reshape(N)


# speedup vs baseline: 1.1769x; 1.1769x over previous
import functools

import jax
import jax.numpy as jnp
from jax import lax
from jax.experimental import pallas as pl
from jax.experimental.pallas import tpu as pltpu
from jax.experimental.pallas import tpu_sc as plsc

F32 = jnp.float32
I32 = jnp.int32

KS = 5
KK = KS * KS
OUT = 16
N = 10000
E = 160000
NC = 2
NS = 16
NW = NC * NS
CH = 128
NCHUNK = 40
PE = CH * NCHUNK * NW
NCH0 = 46
NCH1 = 34
NCHMAX = max(NCH0, NCH1)
POUT = 2 * OUT
HW = KK * POUT
NPAD = 10112
ZR = NPAD // NS
NB = 2000


def _edge_prep_body(attr_ref, ei_ref, basis_ref, gidx_ref):
    a0 = attr_ref[0, :]
    a1 = attr_ref[1, :]
    v0 = a0 * float(KS - 1)
    v1 = a1 * float(KS - 1)
    lo0 = jnp.floor(v0)
    lo1 = jnp.floor(v1)
    f0 = v0 - lo0
    f1 = v1 - lo1
    lo0i = jnp.clip(lo0.astype(I32), 0, KS - 1)
    lo1i = jnp.clip(lo1.astype(I32), 0, KS - 1)
    hi0i = jnp.clip(lo0i + 1, 0, KS - 1)
    hi1i = jnp.clip(lo1i + 1, 0, KS - 1)
    g0 = f0
    g1 = f1
    src = ei_ref[0, :]
    del hi0i
    basis_ref[0, :] = (1.0 - g0) * (1.0 - g1)
    basis_ref[1, :] = g0 * (1.0 - g1)
    basis_ref[2, :] = (1.0 - g0) * g1
    basis_ref[3, :] = g0 * g1
    gidx_ref[0, :] = src * KK + lo0i + KS * lo1i
    gidx_ref[1, :] = src * KK + lo0i + KS * hi1i


def _edge_prep(attr_t, ei_pad):
    be = 8192
    grid = PE // be
    return pl.pallas_call(
        _edge_prep_body,
        grid=(grid,),
        in_specs=[
            pl.BlockSpec((2, be), lambda i: (0, i)),
            pl.BlockSpec((2, be), lambda i: (0, i)),
        ],
        out_specs=[
            pl.BlockSpec((4, be), lambda i: (0, i)),
            pl.BlockSpec((2, be), lambda i: (0, i)),
        ],
        out_shape=[
            jax.ShapeDtypeStruct((4, PE), F32),
            jax.ShapeDtypeStruct((2, PE), I32),
        ],
    )(attr_t, ei_pad)


def _h_body(x_ref, w_ref, o_ref):
    o_ref[...] = jnp.dot(x_ref[...], w_ref[...], preferred_element_type=F32)


def _h_matmul(x, wflat):
    din = x.shape[1]
    return pl.pallas_call(
        _h_body,
        grid=(N // NB,),
        in_specs=[
            pl.BlockSpec((NB, din), lambda i: (i, 0)),
            pl.BlockSpec((din, HW), lambda i: (0, 0)),
        ],
        out_specs=pl.BlockSpec((NB, HW), lambda i: (i, 0)),
        out_shape=jax.ShapeDtypeStruct((N, HW), F32),
    )(x, wflat)


def _epi_body(p_ref, d_ref, x_ref, root_ref, b_ref, wn_ref, xo_ref, h_ref):
    agg = p_ref[0] + p_ref[1]
    deg = d_ref[0, :, 0:1] + d_ref[1, :, 0:1]
    agg = agg / jnp.maximum(deg, 1.0)
    lin = jnp.dot(x_ref[...], root_ref[...], preferred_element_type=F32)
    xo = jnp.maximum(agg + lin + b_ref[...], 0.0)
    xo_ref[...] = xo
    h_ref[...] = jnp.dot(xo, wn_ref[...], preferred_element_type=F32)


def _epilogue(p, d, x, root, b, wnext):
    din = x.shape[1]
    return pl.pallas_call(
        _epi_body,
        grid=(N // NB,),
        in_specs=[
            pl.BlockSpec((2, NB, OUT), lambda i: (0, i, 0)),
            pl.BlockSpec((2, NB, OUT), lambda i: (0, i, 0)),
            pl.BlockSpec((NB, din), lambda i: (i, 0)),
            pl.BlockSpec((din, OUT), lambda i: (0, 0)),
            pl.BlockSpec((1, OUT), lambda i: (0, 0)),
            pl.BlockSpec((OUT, HW), lambda i: (0, 0)),
        ],
        out_specs=[
            pl.BlockSpec((NB, OUT), lambda i: (i, 0)),
            pl.BlockSpec((NB, HW), lambda i: (i, 0)),
        ],
        out_shape=[
            jax.ShapeDtypeStruct((N, OUT), F32),
            jax.ShapeDtypeStruct((N, HW), F32),
        ],
    )(p, d, x, root, b.reshape(1, OUT), wnext)


def _final_body(p_ref, d_ref, x_ref, root_ref, b_ref, fcw_ref, fcb_ref, o_ref):
    agg = p_ref[0] + p_ref[1]
    deg = d_ref[0, :, 0:1] + d_ref[1, :, 0:1]
    agg = agg / jnp.maximum(deg, 1.0)
    lin = jnp.dot(x_ref[...], root_ref[...], preferred_element_type=F32)
    h = jnp.maximum(agg + lin + b_ref[...], 0.0)
    o = jnp.dot(h, fcw_ref[...], preferred_element_type=F32) + fcb_ref[...]
    o_ref[...] = jax.nn.sigmoid(o)


def _final(p, d, x, root, b, fc_w, fc_b):
    return pl.pallas_call(
        _final_body,
        grid=(N // NB,),
        in_specs=[
            pl.BlockSpec((2, NB, OUT), lambda i: (0, i, 0)),
            pl.BlockSpec((2, NB, OUT), lambda i: (0, i, 0)),
            pl.BlockSpec((NB, OUT), lambda i: (i, 0)),
            pl.BlockSpec((OUT, OUT), lambda i: (0, 0)),
            pl.BlockSpec((1, OUT), lambda i: (0, 0)),
            pl.BlockSpec((OUT, 1), lambda i: (0, 0)),
            pl.BlockSpec((1, 1), lambda i: (0, 0)),
        ],
        out_specs=pl.BlockSpec((NB, 1), lambda i: (i, 0)),
        out_shape=jax.ShapeDtypeStruct((N, 1), F32),
    )(p, d, x, root, b.reshape(1, OUT), fc_w, fc_b.reshape(1, 1))


@functools.cache
def _make_sc_pass(with_deg):
    mesh = plsc.VectorSubcoreMesh(core_axis_name="c", subcore_axis_name="s",
                                  num_cores=NC, num_subcores=NS)
    n_out = 2 if with_deg else 1
    out_type = [jax.ShapeDtypeStruct((NC, NPAD, OUT), F32)] * n_out
    scratch = [
        pltpu.VMEM_SHARED((NPAD, OUT), F32),
        pltpu.VMEM((NCHMAX, CH), I32),
        pltpu.VMEM((2, NCHMAX, CH), I32),
        pltpu.VMEM((4, NCHMAX, CH), F32),
        pltpu.VMEM((2, 2, CH, POUT), F32),
        pltpu.VMEM((2, CH, OUT), F32),
        pltpu.VMEM((ZR, OUT), F32),
        pltpu.SemaphoreType.DMA,
        pltpu.SemaphoreType.DMA,
        pltpu.SemaphoreType.DMA,
        pltpu.SemaphoreType.DMA,
    ]
    if with_deg:
        scratch.insert(1, pltpu.VMEM_SHARED((NPAD, OUT), F32))
        scratch.append(pltpu.VMEM((CH, OUT), F32))

    def body(*refs):
        if with_deg:
            (h_hbm, gidx_hbm, basis_hbm, dst_hbm, out_hbm, deg_hbm,
             acc_sh, deg_sh, dst_all, gidx_all, basis_all,
             rbuf, msg_v, zbuf, g0, g1, w0, w1, ones_v) = refs
        else:
            (h_hbm, gidx_hbm, basis_hbm, dst_hbm, out_hbm,
             acc_sh, dst_all, gidx_all, basis_all,
             rbuf, msg_v, zbuf, g0, g1, w0, w1) = refs
            deg_hbm = deg_sh = ones_v = None

        c = lax.axis_index("c")
        s = lax.axis_index("s")
        base_chunk = jnp.where(c == 0, s * NCH0, NS * NCH0 + s * NCH1)
        my_npair = jnp.where(c == 0, NCH0 // 2, NCH1 // 2)
        zero_row = jnp.zeros((OUT,), F32)

        def zero_body(j, carry):
            zbuf[j] = zero_row
            return carry

        lax.fori_loop(0, ZR, zero_body, 0)
        r0off = s * ZR
        pltpu.sync_copy(zbuf, acc_sh.at[pl.ds(r0off, ZR)])
        if with_deg:
            pltpu.sync_copy(zbuf, deg_sh.at[pl.ds(r0off, ZR)])
            one_row = jnp.ones((OUT,), F32)

            def one_body(j, carry):
                ones_v[j] = one_row
                return carry

            lax.fori_loop(0, CH, one_body, 0)

        pltpu.sync_copy(dst_hbm.at[pl.ds(base_chunk, NCHMAX)], dst_all)
        for si in range(2):
            pltpu.sync_copy(gidx_hbm.at[si, pl.ds(base_chunk, NCHMAX)],
                            gidx_all.at[si])
        for si in range(4):
            pltpu.sync_copy(basis_hbm.at[si, pl.ds(base_chunk, NCHMAX)],
                            basis_all.at[si])

        plsc.subcore_barrier()

        gsems = (g0, g1)
        wsems = (w0, w1)
        dummy = h_hbm.at[pl.ds(0, CH)]
        dummy16 = out_hbm.at[0, pl.ds(0, CH)]

        def fire_gather(p, cix):
            for si in range(2):
                pltpu.async_copy(h_hbm.at[gidx_all.at[si, cix]],
                                 rbuf.at[p, si], gsems[p])

        def drain_gather(p):
            for si in range(2):
                pltpu.make_async_copy(dummy, rbuf.at[p, si], gsems[p]).wait()

        def fire_scatter(p, cix):
            pltpu.async_copy(msg_v.at[p], acc_sh.at[dst_all.at[cix]],
                             wsems[p], add=True)
            if with_deg:
                pltpu.async_copy(ones_v, deg_sh.at[dst_all.at[cix]],
                                 wsems[p], add=True)

        def drain_scatter(p):
            pltpu.make_async_copy(dummy16, msg_v.at[p], wsems[p]).wait()
            if with_deg:
                pltpu.make_async_copy(dummy16, ones_v, wsems[p]).wait()

        def compute(p, cix):
            def comp(gi, carry2):
                j0 = gi * 16
                bvs = [basis_all[si, cix, pl.ds(j0, 16)] for si in range(4)]
                for jj in range(16):
                    j = j0 + jj
                    m = (bvs[0][jj] * rbuf[p, 0, j, pl.ds(0, 16)]
                         + bvs[1][jj] * rbuf[p, 0, j, pl.ds(16, 16)]
                         + bvs[2][jj] * rbuf[p, 1, j, pl.ds(0, 16)]
                         + bvs[3][jj] * rbuf[p, 1, j, pl.ds(16, 16)])
                    msg_v[p, j] = m
                return carry2

            lax.fori_loop(0, CH // 16, comp, 0)

        fire_gather(0, 0)

        def pair_body(gg, carry):
            a = 2 * gg
            b = a + 1
            fire_gather(1, b)
            drain_gather(0)

            @pl.when(gg >= 1)
            def _():
                drain_scatter(0)

            compute(0, a)
            fire_scatter(0, a)

            @pl.when(gg < my_npair - 1)
            def _():
                fire_gather(0, a + 2)

            drain_gather(1)

            @pl.when(gg >= 1)
            def _():
                drain_scatter(1)

            compute(1, b)
            fire_scatter(1, b)
            return carry

        lax.fori_loop(0, my_npair, pair_body, 0)
        drain_scatter(0)
        drain_scatter(1)

        plsc.subcore_barrier()

        pltpu.sync_copy(acc_sh.at[pl.ds(r0off, ZR)], zbuf)
        pltpu.sync_copy(zbuf, out_hbm.at[c, pl.ds(r0off, ZR)])
        if with_deg:
            pltpu.sync_copy(deg_sh.at[pl.ds(r0off, ZR)], zbuf)
            pltpu.sync_copy(zbuf, deg_hbm.at[c, pl.ds(r0off, ZR)])

    return pl.kernel(body, out_type=out_type, mesh=mesh,
                     scratch_types=scratch,
                     compiler_params=pltpu.CompilerParams(
                         use_tc_tiling_on_sc=False))


def kernel(x, edge_index, edge_attr, W1, root1, b1, W2, root2, b2,
           W3, root3, b3, fc_w, fc_b):
    src = edge_index[0]
    dst = edge_index[1]
    attr_t = jnp.pad(edge_attr.T, ((0, 0), (0, PE - E)))
    trash = N + (jnp.arange(PE - E, dtype=I32) % (NPAD - N))
    ei_pad = jnp.stack([
        jnp.pad(src, (0, PE - E)),
        jnp.concatenate([dst, trash]),
    ])
    basis4, gidx4 = _edge_prep(attr_t, ei_pad)
    dst3 = ei_pad[1].reshape(PE // CH, CH)
    gidx3 = gidx4.reshape(2, PE // CH, CH)
    basis3 = basis4.reshape(4, PE // CH, CH)

    def pair_flat(W):
        wn = jnp.concatenate([W[1:], jnp.zeros_like(W[:1])], axis=0)
        wp = jnp.concatenate([W, wn], axis=2)
        return jnp.transpose(wp, (1, 0, 2)).reshape(W.shape[1], HW)

    wf1 = pair_flat(W1)
    wf2 = pair_flat(W2)
    wf3 = pair_flat(W3)

    h1 = _h_matmul(x, wf1).reshape(N * KK, POUT)
    p1, dcount = _make_sc_pass(True)(h1, gidx3, basis3, dst3)
    x1, h2 = _epilogue(p1, dcount, x, root1, b1, wf2)
    p2 = _make_sc_pass(False)(h2.reshape(N * KK, POUT), gidx3, basis3, dst3)
    p2 = p2[0] if isinstance(p2, (list, tuple)) else p2
    x2, h3 = _epilogue(p2, dcount, x1, root2, b2, wf3)
    p3 = _make_sc_pass(False)(h3.reshape(N * KK, POUT), gidx3, basis3, dst3)
    p3 = p3[0] if isinstance(p3, (list, tuple)) else p3
    out = _final(p3, dcount, x2, root3, b3, fc_w, fc_b)
    return out.reshape(N)
